# baseline (speedup 1.0000x reference)
import functools

import jax
import jax.numpy as jnp
from jax import lax
from jax.experimental import pallas as pl
from jax.experimental.pallas import tpu as pltpu
from jax.experimental.pallas import tpu_sc as plsc

_NUM_CORES = 2
_NUM_SUBCORES = 16
_LANES = 16
_BIG = 2**30


def _sc_argmax_gather(s5, x5, s_sz, b_sz, n_sz, d):
    n_chunks = n_sz // 128
    p_per_w = s_sz * b_sz // (_NUM_CORES * _NUM_SUBCORES)
    mesh = plsc.VectorSubcoreMesh(core_axis_name="c", subcore_axis_name="s")

    @functools.partial(
        pl.kernel,
        mesh=mesh,
        compiler_params=pltpu.CompilerParams(needs_layout_passes=False),
        out_type=jax.ShapeDtypeStruct((b_sz, s_sz, d), jnp.float32),
        scratch_types=[
            pltpu.VMEM((2, n_chunks, 128), jnp.float32),
            pltpu.VMEM((n_chunks, _LANES), jnp.float32),
            pltpu.VMEM((2, d // 8, 8, 128), jnp.float32),
            pltpu.VMEM((p_per_w, d), jnp.float32),
            pltpu.SemaphoreType.DMA,
            pltpu.SemaphoreType.DMA,
            pltpu.SemaphoreType.DMA,
            pltpu.SemaphoreType.DMA,
        ],
    )
    def k(s5_hbm, x5_hbm, out_hbm, row_v, segmax_v, win_v, col_v, s0, s1, ws, os):
        wid = lax.axis_index("s") * _NUM_CORES + lax.axis_index("c")
        base = wid * p_per_w
        ssems = (s0, s1)
        lanes16 = lax.broadcasted_iota(jnp.int32, (_LANES,), 0)

        def row_dma(j):
            p = base + j
            s_idx = p // b_sz
            b_idx = p % b_sz
            return pltpu.async_copy(
                s5_hbm.at[s_idx, b_idx >> 3, :, b_idx & 7, :],
                row_v.at[j % 2],
                ssems[j % 2],
            )

        sdmas = [row_dma(0), row_dma(1)]

        win_dmas = []
        lane_tgts = []
        out_dmas = []

        def finish(j):
            p = base + j
            win_dmas[j].wait()
            for g_ in range(d // _LANES):
                rows = g_ * _LANES + lanes16
                col_v[j, pl.ds(g_ * _LANES, _LANES)] = plsc.load_gather(
                    win_v,
                    [jnp.full((_LANES,), j % 2, jnp.int32), rows >> 3, rows & 7, lane_tgts[j]],
                )
            out_dmas.append(
                pltpu.async_copy(col_v.at[j], out_hbm.at[p % b_sz, p // b_sz], os)
            )

        for j in range(p_per_w):
            b_idx = (base + j) % b_sz
            jb = j % 2
            sdmas[j].wait()

            def pass_a(i, g):
                m = row_v[jb, i, pl.ds(0, _LANES)]
                for kk in range(1, 128 // _LANES):
                    m = jnp.maximum(m, row_v[jb, i, pl.ds(kk * _LANES, _LANES)])
                segmax_v[i] = m
                return jnp.maximum(g, m)

            g = lax.fori_loop(
                0, n_chunks, pass_a, jnp.full((_LANES,), -jnp.inf, jnp.float32)
            )
            m_scalar = jnp.max(g)
            m_vec = jnp.full((_LANES,), m_scalar, jnp.float32)

            def find_i(i, fv):
                eq = segmax_v[i] == m_vec
                iv = jnp.full((_LANES,), i, jnp.int32)
                return jnp.minimum(fv, jnp.where(eq, iv, jnp.int32(_BIG)))

            fv = lax.fori_loop(
                0, n_chunks, find_i, jnp.full((_LANES,), _BIG, jnp.int32)
            )
            i_tgt = jnp.min(fv)

            cbest = jnp.full((_LANES,), _BIG, jnp.int32)
            for kk in range(128 // _LANES):
                v = row_v[jb, i_tgt, pl.ds(kk * _LANES, _LANES)]
                cpos = kk * _LANES + lanes16
                cbest = jnp.minimum(cbest, jnp.where(v == m_vec, cpos, jnp.int32(_BIG)))
            c_tgt = jnp.min(cbest)
            lane_tgts.append(jnp.full((_LANES,), c_tgt, jnp.int32))

            if j + 2 < p_per_w:
                sdmas.append(row_dma(j + 2))

            win_dmas.append(
                pltpu.async_copy(x5_hbm.at[b_idx, :, i_tgt, :, :], win_v.at[j % 2], ws)
            )
            finish(j)

        for dma in out_dmas:
            dma.wait()

    return k(s5, x5)


def kernel(X, samples, num_samples):
    b, n, d = X.shape
    s = samples.shape[0]
    s5 = samples.reshape(s, b // 8, 8, n // 128, 128).transpose(0, 1, 3, 2, 4)
    x5 = (
        X.transpose(0, 2, 1)
        .reshape(b, d // 8, 8, n // 128, 128)
        .transpose(0, 1, 3, 2, 4)
    )
    return _sc_argmax_gather(s5, x5, s, b, n, d)


# speedup vs baseline: 1.0010x; 1.0010x over previous
import functools

import jax
import jax.numpy as jnp
from jax import lax
from jax.experimental import pallas as pl
from jax.experimental.pallas import tpu as pltpu
from jax.experimental.pallas import tpu_sc as plsc

_NUM_CORES = 2
_NUM_SUBCORES = 16
_LANES = 16
_BIG = 2**30


def _sc_argmax_gather(s5, x5, s_sz, b_sz, n_sz, d):
    n_chunks = n_sz // 128
    p_per_w = s_sz * b_sz // (_NUM_CORES * _NUM_SUBCORES)
    mesh = plsc.VectorSubcoreMesh(core_axis_name="c", subcore_axis_name="s")

    @functools.partial(
        pl.kernel,
        mesh=mesh,
        compiler_params=pltpu.CompilerParams(needs_layout_passes=False),
        out_type=jax.ShapeDtypeStruct((b_sz, s_sz, d), jnp.float32),
        scratch_types=[
            pltpu.VMEM((2, n_chunks, 128), jnp.float32),
            pltpu.VMEM((n_chunks, _LANES), jnp.float32),
            pltpu.VMEM((2, d // 8, 8, 128), jnp.float32),
            pltpu.VMEM((p_per_w, d), jnp.float32),
            pltpu.SemaphoreType.DMA,
            pltpu.SemaphoreType.DMA,
            pltpu.SemaphoreType.DMA,
            pltpu.SemaphoreType.DMA,
        ],
    )
    def k(s5_hbm, x5_hbm, out_hbm, row_v, segmax_v, win_v, col_v, s0, s1, ws, os):
        wid = lax.axis_index("s") * _NUM_CORES + lax.axis_index("c")
        base = wid * p_per_w
        ssems = (s0, s1)
        lanes16 = lax.broadcasted_iota(jnp.int32, (_LANES,), 0)

        def row_dma(j):
            p = base + j
            s_idx = p // b_sz
            b_idx = p % b_sz
            return pltpu.async_copy(
                s5_hbm.at[s_idx, b_idx >> 3, :, b_idx & 7, :],
                row_v.at[j % 2],
                ssems[j % 2],
            )

        sdmas = [row_dma(0), row_dma(1)]

        win_dmas = []
        lane_tgts = []
        out_dmas = []

        def finish(j):
            p = base + j
            win_dmas[j].wait()
            for g_ in range(d // _LANES):
                rows = g_ * _LANES + lanes16
                col_v[j, pl.ds(g_ * _LANES, _LANES)] = plsc.load_gather(
                    win_v,
                    [jnp.full((_LANES,), j % 2, jnp.int32), rows >> 3, rows & 7, lane_tgts[j]],
                )
            out_dmas.append(
                pltpu.async_copy(col_v.at[j], out_hbm.at[p % b_sz, p // b_sz], os)
            )

        for j in range(p_per_w):
            b_idx = (base + j) % b_sz
            jb = j % 2
            sdmas[j].wait()

            def pass_a(i, g):
                m = row_v[jb, i, pl.ds(0, _LANES)]
                for kk in range(1, 128 // _LANES):
                    m = jnp.maximum(m, row_v[jb, i, pl.ds(kk * _LANES, _LANES)])
                segmax_v[i] = m
                return jnp.maximum(g, m)

            g = lax.fori_loop(
                0, n_chunks, pass_a,
                jnp.full((_LANES,), -jnp.inf, jnp.float32), unroll=8,
            )
            m_scalar = jnp.max(g)
            m_vec = jnp.full((_LANES,), m_scalar, jnp.float32)

            def find_i(i, fv):
                eq = segmax_v[i] == m_vec
                iv = jnp.full((_LANES,), i, jnp.int32)
                return jnp.minimum(fv, jnp.where(eq, iv, jnp.int32(_BIG)))

            fv = lax.fori_loop(
                0, n_chunks, find_i,
                jnp.full((_LANES,), _BIG, jnp.int32), unroll=8,
            )
            i_tgt = jnp.min(fv)

            cbest = jnp.full((_LANES,), _BIG, jnp.int32)
            for kk in range(128 // _LANES):
                v = row_v[jb, i_tgt, pl.ds(kk * _LANES, _LANES)]
                cpos = kk * _LANES + lanes16
                cbest = jnp.minimum(cbest, jnp.where(v == m_vec, cpos, jnp.int32(_BIG)))
            c_tgt = jnp.min(cbest)
            lane_tgts.append(jnp.full((_LANES,), c_tgt, jnp.int32))

            if j + 2 < p_per_w:
                sdmas.append(row_dma(j + 2))

            win_dmas.append(
                pltpu.async_copy(x5_hbm.at[b_idx, :, i_tgt, :, :], win_v.at[j % 2], ws)
            )
            finish(j)

        for dma in out_dmas:
            dma.wait()

    return k(s5, x5)


def kernel(X, samples, num_samples):
    b, n, d = X.shape
    s = samples.shape[0]
    s5 = samples.reshape(s, b // 8, 8, n // 128, 128).transpose(0, 1, 3, 2, 4)
    x5 = (
        X.transpose(0, 2, 1)
        .reshape(b, d // 8, 8, n // 128, 128)
        .transpose(0, 1, 3, 2, 4)
    )
    return _sc_argmax_gather(s5, x5, s, b, n, d)


# speedup vs baseline: 1.4143x; 1.4129x over previous
import functools

import jax
import jax.numpy as jnp
from jax import lax
from jax.experimental import pallas as pl
from jax.experimental.pallas import tpu as pltpu
from jax.experimental.pallas import tpu_sc as plsc

_NUM_CORES = 2
_NUM_SUBCORES = 16
_LANES = 16


def _argmax_body(x_ref, out_ref, m_ref, a_ref):
    i = pl.program_id(0)
    nc = pl.num_programs(0)
    x = x_ref[...]
    s, b, c = x.shape
    gidx = i * c + lax.broadcasted_iota(jnp.int32, x.shape, 2)
    cmax = jnp.max(x, axis=-1)
    masked = jnp.where(x == cmax[..., None], gidx, jnp.int32(2**30))
    carg = jnp.min(masked, axis=-1)

    @pl.when(i == 0)
    def _():
        m_ref[...] = cmax
        a_ref[...] = carg

    @pl.when(i > 0)
    def _():
        upd = cmax > m_ref[...]
        m_ref[...] = jnp.where(upd, cmax, m_ref[...])
        a_ref[...] = jnp.where(upd, carg, a_ref[...])

    @pl.when(i == nc - 1)
    def _():
        pad = jnp.zeros((s, 128 - b), jnp.int32)
        out_ref[...] = jnp.concatenate([a_ref[...], pad], axis=1)


def _argmax_indices(samples, chunk=8192):
    s, b, n = samples.shape
    grid = n // chunk
    return pl.pallas_call(
        _argmax_body,
        grid=(grid,),
        in_specs=[pl.BlockSpec((s, b, chunk), lambda i: (0, 0, i))],
        out_specs=pl.BlockSpec((s, 128), lambda i: (0, 0)),
        out_shape=jax.ShapeDtypeStruct((s, 128), jnp.int32),
        scratch_shapes=[
            pltpu.VMEM((s, b), jnp.float32),
            pltpu.VMEM((s, b), jnp.int32),
        ],
    )(samples)


def _sc_gather(x5, idx, s_sz, b_sz, d):
    n_workers = _NUM_SUBCORES
    b_per_w = b_sz * s_sz // n_workers
    mesh = plsc.VectorSubcoreMesh(
        core_axis_name="c", subcore_axis_name="s", num_cores=1
    )

    @functools.partial(
        pl.kernel,
        mesh=mesh,
        compiler_params=pltpu.CompilerParams(needs_layout_passes=False),
        out_type=jax.ShapeDtypeStruct((b_sz, s_sz, d), jnp.float32),
        scratch_types=[
            pltpu.VMEM((_LANES,), jnp.int32),
            pltpu.VMEM((b_per_w, d // 8, 8, 128), jnp.float32),
            pltpu.VMEM((b_per_w, d), jnp.float32),
            pltpu.SemaphoreType.DMA,
            pltpu.SemaphoreType.DMA,
        ],
    )
    def gather_kernel(x5_hbm, idx_hbm, out_hbm, idx_v, win_v, col_v, wsem, osem):
        wid = lax.axis_index("s")
        base = wid * b_per_w
        s_idx = base // b_sz
        pltpu.sync_copy(idx_hbm.at[s_idx, pl.ds(0, _LANES)], idx_v)
        n_vecs = []
        win_dmas = []
        for j in range(b_per_w):
            b_idx = base % b_sz + j
            n_vec = plsc.load_gather(idx_v, [jnp.full((_LANES,), b_idx, jnp.int32)])
            n_hi = n_vec[0] >> 7
            n_vecs.append(n_vec)
            win_dmas.append(
                pltpu.async_copy(x5_hbm.at[b_idx, :, n_hi, :, :], win_v.at[j], wsem)
            )
        lanes = lax.broadcasted_iota(jnp.int32, (_LANES,), 0)
        out_dmas = []
        for j in range(b_per_w):
            b_idx = base % b_sz + j
            win_dmas[j].wait()
            for g in range(d // _LANES):
                rows = g * _LANES + lanes
                col_v[j, pl.ds(g * _LANES, _LANES)] = plsc.load_gather(
                    win_v, [jnp.full((_LANES,), j, jnp.int32), rows >> 3, rows & 7, n_vecs[j] & 127]
                )
            out_dmas.append(
                pltpu.async_copy(col_v.at[j], out_hbm.at[b_idx, s_idx], osem)
            )
        for dma in out_dmas:
            dma.wait()

    return gather_kernel(x5, idx)


def kernel(X, samples, num_samples):
    b, n, d = X.shape
    s = samples.shape[0]
    idx = _argmax_indices(samples)
    x5 = (
        X.transpose(0, 2, 1)
        .reshape(b, d // 8, 8, n // 128, 128)
        .transpose(0, 1, 3, 2, 4)
    )
    return _sc_gather(x5, idx, s, b, d)


# speedup vs baseline: 1.4277x; 1.0095x over previous
import functools

import jax
import jax.numpy as jnp
from jax import lax
from jax.experimental import pallas as pl
from jax.experimental.pallas import tpu as pltpu
from jax.experimental.pallas import tpu_sc as plsc

_NUM_CORES = 2
_NUM_SUBCORES = 16
_LANES = 16


def _argmax_body(x_ref, out_ref, m_ref, a_ref):
    i = pl.program_id(0)
    nc = pl.num_programs(0)
    x = x_ref[...]
    s, b, c = x.shape
    gidx = i * c + lax.broadcasted_iota(jnp.int32, x.shape, 2)
    cmax = jnp.max(x, axis=-1)
    masked = jnp.where(x == cmax[..., None], gidx, jnp.int32(2**30))
    carg = jnp.min(masked, axis=-1)

    @pl.when(i == 0)
    def _():
        m_ref[...] = cmax
        a_ref[...] = carg

    @pl.when(i > 0)
    def _():
        upd = cmax > m_ref[...]
        m_ref[...] = jnp.where(upd, cmax, m_ref[...])
        a_ref[...] = jnp.where(upd, carg, a_ref[...])

    @pl.when(i == nc - 1)
    def _():
        pad = jnp.zeros((s, 128 - b), jnp.int32)
        out_ref[...] = jnp.concatenate([a_ref[...], pad], axis=1)


def _argmax_indices(samples, chunk=16384):
    s, b, n = samples.shape
    grid = n // chunk
    return pl.pallas_call(
        _argmax_body,
        grid=(grid,),
        in_specs=[pl.BlockSpec((s, b, chunk), lambda i: (0, 0, i))],
        out_specs=pl.BlockSpec((s, 128), lambda i: (0, 0)),
        out_shape=jax.ShapeDtypeStruct((s, 128), jnp.int32),
        scratch_shapes=[
            pltpu.VMEM((s, b), jnp.float32),
            pltpu.VMEM((s, b), jnp.int32),
        ],
    )(samples)


def _sc_gather(x5, idx, s_sz, b_sz, d):
    n_workers = _NUM_SUBCORES
    b_per_w = b_sz * s_sz // n_workers
    mesh = plsc.VectorSubcoreMesh(
        core_axis_name="c", subcore_axis_name="s", num_cores=1
    )

    @functools.partial(
        pl.kernel,
        mesh=mesh,
        compiler_params=pltpu.CompilerParams(needs_layout_passes=False),
        out_type=jax.ShapeDtypeStruct((b_sz, s_sz, d), jnp.float32),
        scratch_types=[
            pltpu.VMEM((_LANES,), jnp.int32),
            pltpu.VMEM((b_per_w, d // 8, 8, 128), jnp.float32),
            pltpu.VMEM((b_per_w, d), jnp.float32),
            pltpu.SemaphoreType.DMA,
            pltpu.SemaphoreType.DMA,
        ],
    )
    def gather_kernel(x5_hbm, idx_hbm, out_hbm, idx_v, win_v, col_v, wsem, osem):
        wid = lax.axis_index("s")
        base = wid * b_per_w
        s_idx = base // b_sz
        pltpu.sync_copy(idx_hbm.at[s_idx, pl.ds(0, _LANES)], idx_v)
        n_vecs = []
        win_dmas = []
        for j in range(b_per_w):
            b_idx = base % b_sz + j
            n_vec = plsc.load_gather(idx_v, [jnp.full((_LANES,), b_idx, jnp.int32)])
            n_hi = n_vec[0] >> 7
            n_vecs.append(n_vec)
            win_dmas.append(
                pltpu.async_copy(x5_hbm.at[b_idx, :, n_hi, :, :], win_v.at[j], wsem)
            )
        lanes = lax.broadcasted_iota(jnp.int32, (_LANES,), 0)
        out_dmas = []
        for j in range(b_per_w):
            b_idx = base % b_sz + j
            win_dmas[j].wait()
            for g in range(d // _LANES):
                rows = g * _LANES + lanes
                col_v[j, pl.ds(g * _LANES, _LANES)] = plsc.load_gather(
                    win_v, [jnp.full((_LANES,), j, jnp.int32), rows >> 3, rows & 7, n_vecs[j] & 127]
                )
            out_dmas.append(
                pltpu.async_copy(col_v.at[j], out_hbm.at[b_idx, s_idx], osem)
            )
        for dma in out_dmas:
            dma.wait()

    return gather_kernel(x5, idx)


def kernel(X, samples, num_samples):
    b, n, d = X.shape
    s = samples.shape[0]
    idx = _argmax_indices(samples)
    x5 = (
        X.transpose(0, 2, 1)
        .reshape(b, d // 8, 8, n // 128, 128)
        .transpose(0, 1, 3, 2, 4)
    )
    return _sc_gather(x5, idx, s, b, d)
